# baseline (speedup 1.0000x reference)
import functools

import jax
import jax.numpy as jnp
from jax.experimental import pallas as pl
from jax.experimental.pallas import tpu as pltpu

N = 4096
D = 128
N_HEADS = 2
KEEP_RATIO = 0.5
INV_TEMP = 2.0
BLK = 512
BIG = 1e6


def _layer_body(x_ref, adj_ref, ori_ref, w_ref, out_ref, hn_ref):
    i = pl.program_id(0)

    @pl.when(i == 0)
    def _():
        x = x_ref[...]
        for h in range(N_HEADS):
            hh = jnp.dot(x, w_ref[h], preferred_element_type=jnp.float32)
            norm = jnp.sqrt(jnp.sum(hh * hh, axis=1, keepdims=True))
            hn_ref[h] = (hh * (jnp.sqrt(INV_TEMP) / (norm + 1e-8))
                         ).astype(jnp.bfloat16)

    adjm = adj_ref[...].astype(jnp.bfloat16)
    acc = jnp.zeros((BLK, D), jnp.float32)
    for h in range(N_HEADS):
        hn = hn_ref[h]
        hnb = hn_ref[h, pl.ds(i * BLK, BLK), :]
        sim = jax.lax.dot_general(
            hnb, hn, (((1,), (1,)), ((), ())),
            preferred_element_type=jnp.float32)
        sim_m = sim.astype(jnp.bfloat16) * adjm
        rmax = jnp.max(sim_m, axis=1, keepdims=True)
        arg = jnp.minimum(
            (sim_m - KEEP_RATIO * rmax) * jnp.bfloat16(BIG) + sim_m, sim_m)
        p = jnp.exp(arg)
        s = jnp.sum(p.astype(jnp.float32), axis=1, keepdims=True)
        agg = jnp.dot(p, ori_ref[...], preferred_element_type=jnp.float32)
        acc = acc + agg / s
    out_ref[...] = acc * (1.0 / N_HEADS)


def _prop_layer(x, adj, ori, w_l):
    grid = (N // BLK,)
    return pl.pallas_call(
        _layer_body,
        grid=grid,
        in_specs=[
            pl.BlockSpec((N, D), lambda i: (0, 0)),
            pl.BlockSpec((BLK, N), lambda i: (i, 0)),
            pl.BlockSpec((N, D), lambda i: (0, 0)),
            pl.BlockSpec((N_HEADS, D, D), lambda i: (0, 0, 0)),
        ],
        out_specs=pl.BlockSpec((BLK, D), lambda i: (i, 0)),
        out_shape=jax.ShapeDtypeStruct((N, D), jnp.float32),
        scratch_shapes=[pltpu.VMEM((N_HEADS, N, D), jnp.bfloat16)],
    )(x, adj, ori, w_l)


@functools.partial(jax.jit, static_argnames=())
def kernel(features, adj0, adj1, W):
    ori_bf = features.astype(jnp.bfloat16)
    x = _prop_layer(features, adj0, ori_bf, W[0])
    x = _prop_layer(x, adj1, ori_bf, W[1])
    return x


# speedup vs baseline: 1.0155x; 1.0155x over previous
import functools

import jax
import jax.numpy as jnp
from jax.experimental import pallas as pl
from jax.experimental.pallas import tpu as pltpu

N = 4096
D = 128
N_HEADS = 2
KEEP_RATIO = 0.5
INV_TEMP = 2.0
BLK = 128
BIG = 1e6


def _layer_body(x_ref, adj_ref, ori_ref, w_ref, out_ref, hn_ref):
    i = pl.program_id(0)

    @pl.when(i == 0)
    def _():
        x = x_ref[...]
        for h in range(N_HEADS):
            hh = jnp.dot(x, w_ref[h], preferred_element_type=jnp.float32)
            norm = jnp.sqrt(jnp.sum(hh * hh, axis=1, keepdims=True))
            hn_ref[h] = (hh * (jnp.sqrt(INV_TEMP) / (norm + 1e-8))
                         ).astype(jnp.bfloat16)

    adjm = adj_ref[...].astype(jnp.bfloat16)
    acc = jnp.zeros((BLK, D), jnp.float32)
    for h in range(N_HEADS):
        hn = hn_ref[h]
        hnb = hn_ref[h, pl.ds(i * BLK, BLK), :]
        sim = jax.lax.dot_general(
            hnb, hn, (((1,), (1,)), ((), ())),
            preferred_element_type=jnp.float32)
        sim_m = sim.astype(jnp.bfloat16) * adjm
        rmax = jnp.max(sim_m, axis=1, keepdims=True)
        arg = jnp.minimum(
            (sim_m - KEEP_RATIO * rmax) * jnp.bfloat16(BIG) + sim_m, sim_m)
        p = jnp.exp(arg)
        s = jnp.sum(p.astype(jnp.float32), axis=1, keepdims=True)
        agg = jnp.dot(p, ori_ref[...], preferred_element_type=jnp.float32)
        acc = acc + agg / s
    out_ref[...] = acc * (1.0 / N_HEADS)


def _prop_layer(x, adj, ori, w_l):
    grid = (N // BLK,)
    return pl.pallas_call(
        _layer_body,
        grid=grid,
        in_specs=[
            pl.BlockSpec((N, D), lambda i: (0, 0)),
            pl.BlockSpec((BLK, N), lambda i: (i, 0)),
            pl.BlockSpec((N, D), lambda i: (0, 0)),
            pl.BlockSpec((N_HEADS, D, D), lambda i: (0, 0, 0)),
        ],
        out_specs=pl.BlockSpec((BLK, D), lambda i: (i, 0)),
        out_shape=jax.ShapeDtypeStruct((N, D), jnp.float32),
        scratch_shapes=[pltpu.VMEM((N_HEADS, N, D), jnp.bfloat16)],
    )(x, adj, ori, w_l)


@functools.partial(jax.jit, static_argnames=())
def kernel(features, adj0, adj1, W):
    ori_bf = features.astype(jnp.bfloat16)
    x = _prop_layer(features, adj0, ori_bf, W[0])
    x = _prop_layer(x, adj1, ori_bf, W[1])
    return x


# speedup vs baseline: 1.2980x; 1.2782x over previous
import functools

import jax
import jax.numpy as jnp
from jax.experimental import pallas as pl
from jax.experimental.pallas import tpu as pltpu

N = 4096
D = 128
N_HEADS = 2
KEEP_RATIO = 0.5
INV_TEMP = 2.0
BLK = 256
BIG = 1e6


def _layer_body(x_ref, adj_ref, ori_ref, w_ref, out_ref, hn_ref):
    i = pl.program_id(0)

    @pl.when(i == 0)
    def _():
        x = x_ref[...]
        for h in range(N_HEADS):
            hh = jnp.dot(x, w_ref[h], preferred_element_type=jnp.float32)
            norm = jnp.sqrt(jnp.sum(hh * hh, axis=1, keepdims=True))
            hn_ref[h] = (hh * (jnp.sqrt(INV_TEMP) / (norm + 1e-8))
                         ).astype(jnp.bfloat16)

    adjm = adj_ref[...].astype(jnp.bfloat16)
    acc = jnp.zeros((BLK, D), jnp.float32)
    for h in range(N_HEADS):
        hn = hn_ref[h]
        hnb = hn_ref[h, pl.ds(i * BLK, BLK), :]
        sim = jax.lax.dot_general(
            hnb, hn, (((1,), (1,)), ((), ())),
            preferred_element_type=jnp.float32)
        sim_m = sim.astype(jnp.bfloat16) * adjm
        e = jnp.exp(sim_m)
        emax = jnp.max(e, axis=1, keepdims=True)
        thr = jnp.sqrt(emax)
        p = jnp.where(e >= thr, e, jnp.bfloat16(0.0))
        s = jnp.sum(p.astype(jnp.float32), axis=1, keepdims=True)
        agg = jnp.dot(p, ori_ref[...], preferred_element_type=jnp.float32)
        acc = acc + agg / s
    out_ref[...] = acc * (1.0 / N_HEADS)


def _prop_layer(x, adj, ori, w_l):
    grid = (N // BLK,)
    return pl.pallas_call(
        _layer_body,
        grid=grid,
        in_specs=[
            pl.BlockSpec((N, D), lambda i: (0, 0)),
            pl.BlockSpec((BLK, N), lambda i: (i, 0)),
            pl.BlockSpec((N, D), lambda i: (0, 0)),
            pl.BlockSpec((N_HEADS, D, D), lambda i: (0, 0, 0)),
        ],
        out_specs=pl.BlockSpec((BLK, D), lambda i: (i, 0)),
        out_shape=jax.ShapeDtypeStruct((N, D), jnp.float32),
        scratch_shapes=[pltpu.VMEM((N_HEADS, N, D), jnp.bfloat16)],
    )(x, adj, ori, w_l)


@functools.partial(jax.jit, static_argnames=())
def kernel(features, adj0, adj1, W):
    ori_bf = features.astype(jnp.bfloat16)
    x = _prop_layer(features, adj0, ori_bf, W[0])
    x = _prop_layer(x, adj1, ori_bf, W[1])
    return x
